# baseline (speedup 1.0000x reference)
import functools

import jax
import jax.numpy as jnp
from jax import lax
from jax.experimental import pallas as pl
from jax.experimental.pallas import tpu as pltpu
from jax.experimental.pallas import tpu_sc as plsc

_N = 10000
_E = 160000
_C = 256
_H = 128
_TD = 128

_EPAD = 163840
_NACC = 10240
_RPT = _NACC // 16
_CH = 128

_BN = 1000
_GRID = _N // _BN

_mesh = plsc.VectorSubcoreMesh(core_axis_name="c", subcore_axis_name="s",
                               num_cores=2, num_subcores=16)


_DST = _EPAD // 32 // _CH


def _deg_body(dstd_hbm, ones_hbm, zeros_hbm, degp_hbm, idx_v, ones_v, zer_v, acc_sh, sem_s):
    cid = lax.axis_index("c")
    sid = lax.axis_index("s")
    r0 = sid * _RPT
    wid = cid * 16 + sid
    pltpu.sync_copy(dstd_hbm.at[wid], idx_v)
    pltpu.sync_copy(zeros_hbm, zer_v)
    pltpu.sync_copy(ones_hbm, ones_v)
    for j in range(_RPT // _CH):
        pltpu.sync_copy(zer_v, acc_sh.at[pl.ds(r0 + j * _CH, _CH)])
    plsc.subcore_barrier()

    def step(i, carry):
        @pl.when(i >= 4)
        def _drain_one():
            pltpu.make_async_copy(zeros_hbm, zer_v, sem_s).wait()

        pltpu.async_copy(ones_v, acc_sh.at[idx_v.at[i]], sem_s, add=True)
        return carry

    lax.fori_loop(0, _DST, step, 0)
    for _ in range(4):
        pltpu.make_async_copy(zeros_hbm, zer_v, sem_s).wait()
    plsc.subcore_barrier()
    pltpu.sync_copy(acc_sh.at[pl.ds(r0, _RPT)], degp_hbm.at[cid, pl.ds(r0, _RPT)])


_deg_call = functools.partial(
    pl.kernel,
    out_type=jax.ShapeDtypeStruct((2, _NACC, _H), jnp.float32),
    mesh=_mesh,
    scratch_types=[
        pltpu.VMEM((_DST, _CH), jnp.int32),
        pltpu.VMEM((_CH, _H), jnp.float32),
        pltpu.VMEM((_CH, _H), jnp.float32),
        pltpu.VMEM_SHARED((_NACC, _H), jnp.float32),
        pltpu.SemaphoreType.DMA,
    ],
)(_deg_body)


_NST = _EPAD // 16 // _CH


def _scat_body(sdx_hbm, y2_hbm, zeros_hbm, acc_hbm,
               sd_v, rows_v, acc_sh, sem_i, sem_g, sem_s):
    cid = lax.axis_index("c")
    sid = lax.axis_index("s")
    r0 = sid * _RPT
    wid = cid * 16 + sid
    pltpu.sync_copy(zeros_hbm, rows_v.at[0])
    for j in range(_RPT // _CH):
        pltpu.sync_copy(rows_v.at[0], acc_sh.at[pl.ds(r0 + j * _CH, _CH)])
    pltpu.async_copy(sdx_hbm.at[wid, 0], sd_v.at[0], sem_i)
    pltpu.async_copy(sdx_hbm.at[wid, 1], sd_v.at[1], sem_i)
    plsc.subcore_barrier()
    pltpu.make_async_copy(sdx_hbm.at[wid, 0], sd_v.at[0], sem_i).wait()
    pltpu.async_copy(y2_hbm.at[sd_v.at[0, 0]], rows_v.at[0], sem_g)

    def step(i, carry):
        cur = lax.rem(i, 2)
        nxt = lax.rem(i + 1, 2)

        @pl.when(i >= 1)
        def _wait_scat():
            pltpu.make_async_copy(zeros_hbm, rows_v.at[nxt], sem_s).wait()

        @pl.when(i < _NST - 2)
        def _next_idx():
            pltpu.async_copy(sdx_hbm.at[wid, i + 2], sd_v.at[lax.rem(i + 2, 3)],
                             sem_i)

        @pl.when(i < _NST - 1)
        def _next_gather():
            pltpu.make_async_copy(sdx_hbm.at[wid, 0], sd_v.at[0], sem_i).wait()
            pltpu.async_copy(y2_hbm.at[sd_v.at[lax.rem(i + 1, 3), 0]],
                             rows_v.at[nxt], sem_g)

        pltpu.make_async_copy(zeros_hbm, rows_v.at[cur], sem_g).wait()
        pltpu.async_copy(rows_v.at[cur], acc_sh.at[sd_v.at[lax.rem(i, 3), 1]],
                         sem_s, add=True)
        return carry

    lax.fori_loop(0, _NST, step, 0)
    pltpu.make_async_copy(zeros_hbm, rows_v.at[(_NST - 1) % 2], sem_s).wait()
    plsc.subcore_barrier()
    pltpu.sync_copy(acc_sh.at[pl.ds(r0, _RPT)], acc_hbm.at[cid, pl.ds(r0, _RPT)])


_scat_call = functools.partial(
    pl.kernel,
    out_type=jax.ShapeDtypeStruct((2, _NACC, _H), jnp.float32),
    mesh=_mesh,
    scratch_types=[
        pltpu.VMEM((3, 2, _CH), jnp.int32),
        pltpu.VMEM((2, _CH, _H), jnp.float32),
        pltpu.VMEM_SHARED((_NACC, _H), jnp.float32),
        pltpu.SemaphoreType.DMA,
        pltpu.SemaphoreType.DMA,
        pltpu.SemaphoreType.DMA,
    ],
)(_scat_body)


def _mm_body(x_ref, gcw_ref, xw_ref):
    xw = jnp.dot(x_ref[...], gcw_ref[...], preferred_element_type=jnp.float32)
    xw_ref[0] = xw[:, :_H]
    xw_ref[1] = xw[:, _H:]


def _mm(x, gc_W):
    return pl.pallas_call(
        _mm_body,
        grid=(_GRID,),
        in_specs=[
            pl.BlockSpec((_BN, _C), lambda i: (i, 0)),
            pl.BlockSpec((_C, _C), lambda i: (0, 0)),
        ],
        out_specs=pl.BlockSpec((2, _BN, _H), lambda i: (0, i, 0)),
        out_shape=jax.ShapeDtypeStruct((2, _N, _H), jnp.float32),
    )(x, gc_W)


def _scale_body(xw_ref, degp_ref, y_ref):
    deg = degp_ref[0, :, 0] + degp_ref[1, :, 0] + 1.0
    dinv = lax.rsqrt(deg)[None, :, None]
    y_ref[...] = xw_ref[...] * dinv


def _scale(xw_sw, degp):
    return pl.pallas_call(
        _scale_body,
        grid=(_GRID,),
        in_specs=[
            pl.BlockSpec((2, _BN, _H), lambda i: (0, i, 0)),
            pl.BlockSpec((2, _BN, _H), lambda i: (0, i, 0)),
        ],
        out_specs=pl.BlockSpec((2, _BN, _H), lambda i: (0, i, 0)),
        out_shape=jax.ShapeDtypeStruct((2, _N, _H), jnp.float32),
    )(xw_sw, degp)


def _final_body(acc_ref, y_ref, degp_ref, t_ref, gcb_ref, tw_ref, tb_ref,
                lw_ref, lb_ref, g_ref, b_ref, out_ref):
    deg = degp_ref[0, :, 0] + degp_ref[1, :, 0] + 1.0
    dinv = lax.rsqrt(deg)[:, None]
    accf = jnp.concatenate([acc_ref[0], acc_ref[1]], axis=1)
    yf = jnp.concatenate([y_ref[0], y_ref[1]], axis=1)
    g = dinv * (accf + yf) + gcb_ref[...]
    g = g * jax.nn.sigmoid(g)
    te = jnp.dot(t_ref[...], tw_ref[...], preferred_element_type=jnp.float32) + tb_ref[...]
    te = te * jax.nn.sigmoid(te)
    h = g + te
    h = jnp.dot(h, lw_ref[...], preferred_element_type=jnp.float32) + lb_ref[...]
    h = h * jax.nn.sigmoid(h)
    mu = jnp.mean(h, axis=1, keepdims=True)
    var = jnp.mean((h - mu) ** 2, axis=1, keepdims=True)
    out_ref[...] = (h - mu) * lax.rsqrt(var + 1e-5) * g_ref[...] + b_ref[...]


def _final(acc, y_sw, degp, t, gc_b, time_W, time_b, lin_W, lin_b, ln2_g, ln2_b):
    row = lambda i: (i, 0)
    fixed = lambda i: (0, 0)
    return pl.pallas_call(
        _final_body,
        grid=(_GRID,),
        in_specs=[
            pl.BlockSpec((2, _BN, _H), lambda i: (0, i, 0)),
            pl.BlockSpec((2, _BN, _H), lambda i: (0, i, 0)),
            pl.BlockSpec((2, _BN, _H), lambda i: (0, i, 0)),
            pl.BlockSpec((_BN, _TD), row),
            pl.BlockSpec((1, _C), fixed),
            pl.BlockSpec((_TD, _C), fixed),
            pl.BlockSpec((1, _C), fixed),
            pl.BlockSpec((_C, _C), fixed),
            pl.BlockSpec((1, _C), fixed),
            pl.BlockSpec((1, _C), fixed),
            pl.BlockSpec((1, _C), fixed),
        ],
        out_specs=pl.BlockSpec((_BN, _C), row),
        out_shape=jax.ShapeDtypeStruct((_N, _C), jnp.float32),
    )(acc, y_sw, degp, t, gc_b, time_W, time_b, lin_W, lin_b, ln2_g, ln2_b)


def kernel(x, edge_index, t, gc_W, gc_b, time_W, time_b, lin_W, lin_b, ln2_g, ln2_b):
    src = edge_index[0]
    dst = edge_index[1]
    npad = _EPAD - _E
    src_pad = jnp.concatenate([src, jnp.zeros((npad,), jnp.int32)])
    dst_pad = jnp.concatenate([dst, jnp.full((npad,), _N, jnp.int32)])
    src2 = jnp.concatenate([src_pad, src_pad + _N]).reshape(32, _NST, _CH)
    dstr = dst_pad.reshape(16, _NST, _CH)
    sdx = jnp.stack([src2, jnp.concatenate([dstr, dstr])], axis=2)

    ones128 = jnp.ones((_CH, _H), jnp.float32)
    zeros128 = jnp.zeros((_CH, _H), jnp.float32)

    dstd = dst_pad.reshape(32, _DST, _CH)
    degp = _deg_call(dstd, ones128, zeros128)
    xw_sw = _mm(x, gc_W)
    y_sw = _scale(xw_sw, degp)
    y2 = y_sw.reshape(2 * _N, _H)
    acc = _scat_call(sdx, y2, zeros128)

    gc_b2 = gc_b.reshape(1, _C)
    time_b2 = time_b.reshape(1, _C)
    lin_b2 = lin_b.reshape(1, _C)
    ln2_g2 = ln2_g.reshape(1, _C)
    ln2_b2 = ln2_b.reshape(1, _C)
    return _final(acc, y_sw, degp, t, gc_b2, time_W, time_b2,
                  lin_W, lin_b2, ln2_g2, ln2_b2)


# speedup vs baseline: 1.1295x; 1.1295x over previous
import functools

import jax
import jax.numpy as jnp
from jax import lax
from jax.experimental import pallas as pl
from jax.experimental.pallas import tpu as pltpu
from jax.experimental.pallas import tpu_sc as plsc

_N = 10000
_E = 160000
_C = 256
_H = 128
_TD = 128

_EPAD = 163840
_NACC = 10240
_RPT = _NACC // 16
_CH = 128

_BN = 1000
_GRID = _N // _BN

_mesh = plsc.VectorSubcoreMesh(core_axis_name="c", subcore_axis_name="s",
                               num_cores=2, num_subcores=16)


_DST = _EPAD // 32 // _CH


def _deg_body(dstd_hbm, ones_hbm, zeros_hbm, degp_hbm, idx_v, ones_v, zer_v, acc_sh, sem_s):
    cid = lax.axis_index("c")
    sid = lax.axis_index("s")
    r0 = sid * _RPT
    wid = cid * 16 + sid
    pltpu.sync_copy(dstd_hbm.at[wid], idx_v)
    pltpu.sync_copy(zeros_hbm, zer_v)
    pltpu.sync_copy(ones_hbm, ones_v)
    for j in range(_RPT // _CH):
        pltpu.sync_copy(zer_v, acc_sh.at[pl.ds(r0 + j * _CH, _CH)])
    plsc.subcore_barrier()

    def step(i, carry):
        @pl.when(i >= 4)
        def _drain_one():
            pltpu.make_async_copy(zeros_hbm, zer_v, sem_s).wait()

        pltpu.async_copy(ones_v, acc_sh.at[idx_v.at[i]], sem_s, add=True)
        return carry

    lax.fori_loop(0, _DST, step, 0)
    for _ in range(4):
        pltpu.make_async_copy(zeros_hbm, zer_v, sem_s).wait()
    plsc.subcore_barrier()
    pltpu.sync_copy(acc_sh.at[pl.ds(r0, _RPT)], degp_hbm.at[cid, pl.ds(r0, _RPT)])


_deg_call = functools.partial(
    pl.kernel,
    out_type=jax.ShapeDtypeStruct((2, _NACC, _H), jnp.float32),
    mesh=_mesh,
    scratch_types=[
        pltpu.VMEM((_DST, _CH), jnp.int32),
        pltpu.VMEM((_CH, _H), jnp.float32),
        pltpu.VMEM((_CH, _H), jnp.float32),
        pltpu.VMEM_SHARED((_NACC, _H), jnp.float32),
        pltpu.SemaphoreType.DMA,
    ],
)(_deg_body)


_NST = _EPAD // 16 // _CH


def _scat_body(sdx_hbm, y2_hbm, zeros_hbm, acc_hbm,
               sd_v, rows_v, acc_sh, sem_i, sem_g, sem_s):
    cid = lax.axis_index("c")
    sid = lax.axis_index("s")
    r0 = sid * _RPT
    wid = cid * 16 + sid
    pltpu.sync_copy(zeros_hbm, rows_v.at[0])
    for j in range(_RPT // _CH):
        pltpu.sync_copy(rows_v.at[0], acc_sh.at[pl.ds(r0 + j * _CH, _CH)])
    pltpu.async_copy(sdx_hbm.at[wid, 0], sd_v.at[0], sem_i)
    pltpu.async_copy(sdx_hbm.at[wid, 1], sd_v.at[1], sem_i)
    plsc.subcore_barrier()
    pltpu.make_async_copy(sdx_hbm.at[wid, 0], sd_v.at[0], sem_i).wait()
    pltpu.async_copy(y2_hbm.at[sd_v.at[0, 0]], rows_v.at[0], sem_g)

    def step(i, carry):
        cur = lax.rem(i, 2)
        nxt = lax.rem(i + 1, 2)

        @pl.when(i >= 1)
        def _wait_scat():
            pltpu.make_async_copy(zeros_hbm, rows_v.at[nxt], sem_s).wait()

        @pl.when(i < _NST - 2)
        def _next_idx():
            pltpu.async_copy(sdx_hbm.at[wid, i + 2], sd_v.at[lax.rem(i + 2, 3)],
                             sem_i)

        @pl.when(i < _NST - 1)
        def _next_gather():
            pltpu.make_async_copy(sdx_hbm.at[wid, 0], sd_v.at[0], sem_i).wait()
            pltpu.async_copy(y2_hbm.at[sd_v.at[lax.rem(i + 1, 3), 0]],
                             rows_v.at[nxt], sem_g)

        pltpu.make_async_copy(zeros_hbm, rows_v.at[cur], sem_g).wait()
        pltpu.async_copy(rows_v.at[cur], acc_sh.at[sd_v.at[lax.rem(i, 3), 1]],
                         sem_s, add=True)
        return carry

    lax.fori_loop(0, _NST, step, 0)
    pltpu.make_async_copy(zeros_hbm, rows_v.at[(_NST - 1) % 2], sem_s).wait()
    plsc.subcore_barrier()
    pltpu.sync_copy(acc_sh.at[pl.ds(r0, _RPT)], acc_hbm.at[cid, pl.ds(r0, _RPT)])


_scat_call = functools.partial(
    pl.kernel,
    out_type=jax.ShapeDtypeStruct((2, _NACC, _H), jnp.float32),
    mesh=_mesh,
    scratch_types=[
        pltpu.VMEM((3, 2, _CH), jnp.int32),
        pltpu.VMEM((2, _CH, _H), jnp.float32),
        pltpu.VMEM_SHARED((_NACC, _H), jnp.float32),
        pltpu.SemaphoreType.DMA,
        pltpu.SemaphoreType.DMA,
        pltpu.SemaphoreType.DMA,
    ],
)(_scat_body)


def _prep_body(x_ref, gcw_ref, degp_ref, y_ref):
    xw = jnp.dot(x_ref[...], gcw_ref[...], preferred_element_type=jnp.float32)
    deg = degp_ref[0, :, 0] + degp_ref[1, :, 0] + 1.0
    y = xw * lax.rsqrt(deg)[:, None]
    y_ref[0] = y[:, :_H]
    y_ref[1] = y[:, _H:]


def _prep(x, gc_W, degp):
    return pl.pallas_call(
        _prep_body,
        grid=(_GRID,),
        in_specs=[
            pl.BlockSpec((_BN, _C), lambda i: (i, 0)),
            pl.BlockSpec((_C, _C), lambda i: (0, 0)),
            pl.BlockSpec((2, _BN, _H), lambda i: (0, i, 0)),
        ],
        out_specs=pl.BlockSpec((2, _BN, _H), lambda i: (0, i, 0)),
        out_shape=jax.ShapeDtypeStruct((2, _N, _H), jnp.float32),
    )(x, gc_W, degp)


def _final_body(acc_ref, y_ref, degp_ref, t_ref, gcb_ref, tw_ref, tb_ref,
                lw_ref, lb_ref, g_ref, b_ref, out_ref):
    deg = degp_ref[0, :, 0] + degp_ref[1, :, 0] + 1.0
    dinv = lax.rsqrt(deg)[:, None]
    accf = jnp.concatenate([acc_ref[0], acc_ref[1]], axis=1)
    yf = jnp.concatenate([y_ref[0], y_ref[1]], axis=1)
    g = dinv * (accf + yf) + gcb_ref[...]
    g = g * jax.nn.sigmoid(g)
    te = jnp.dot(t_ref[...], tw_ref[...], preferred_element_type=jnp.float32) + tb_ref[...]
    te = te * jax.nn.sigmoid(te)
    h = g + te
    h = jnp.dot(h, lw_ref[...], preferred_element_type=jnp.float32) + lb_ref[...]
    h = h * jax.nn.sigmoid(h)
    mu = jnp.mean(h, axis=1, keepdims=True)
    var = jnp.mean((h - mu) ** 2, axis=1, keepdims=True)
    out_ref[...] = (h - mu) * lax.rsqrt(var + 1e-5) * g_ref[...] + b_ref[...]


def _final(acc, y_sw, degp, t, gc_b, time_W, time_b, lin_W, lin_b, ln2_g, ln2_b):
    row = lambda i: (i, 0)
    fixed = lambda i: (0, 0)
    return pl.pallas_call(
        _final_body,
        grid=(_GRID,),
        in_specs=[
            pl.BlockSpec((2, _BN, _H), lambda i: (0, i, 0)),
            pl.BlockSpec((2, _BN, _H), lambda i: (0, i, 0)),
            pl.BlockSpec((2, _BN, _H), lambda i: (0, i, 0)),
            pl.BlockSpec((_BN, _TD), row),
            pl.BlockSpec((1, _C), fixed),
            pl.BlockSpec((_TD, _C), fixed),
            pl.BlockSpec((1, _C), fixed),
            pl.BlockSpec((_C, _C), fixed),
            pl.BlockSpec((1, _C), fixed),
            pl.BlockSpec((1, _C), fixed),
            pl.BlockSpec((1, _C), fixed),
        ],
        out_specs=pl.BlockSpec((_BN, _C), row),
        out_shape=jax.ShapeDtypeStruct((_N, _C), jnp.float32),
    )(acc, y_sw, degp, t, gc_b, time_W, time_b, lin_W, lin_b, ln2_g, ln2_b)


def kernel(x, edge_index, t, gc_W, gc_b, time_W, time_b, lin_W, lin_b, ln2_g, ln2_b):
    src = edge_index[0]
    dst = edge_index[1]
    npad = _EPAD - _E
    src_pad = jnp.concatenate([src, jnp.zeros((npad,), jnp.int32)])
    dst_pad = jnp.concatenate([dst, jnp.full((npad,), _N, jnp.int32)])
    src2 = jnp.concatenate([src_pad, src_pad + _N]).reshape(32, _NST, _CH)
    dstr = dst_pad.reshape(16, _NST, _CH)
    sdx = jnp.stack([src2, jnp.concatenate([dstr, dstr])], axis=2)

    ones128 = jnp.ones((_CH, _H), jnp.float32)
    zeros128 = jnp.zeros((_CH, _H), jnp.float32)

    dstd = dst_pad.reshape(32, _DST, _CH)
    degp = _deg_call(dstd, ones128, zeros128)
    y_sw = _prep(x, gc_W, degp)
    y2 = y_sw.reshape(2 * _N, _H)
    acc = _scat_call(sdx, y2, zeros128)

    gc_b2 = gc_b.reshape(1, _C)
    time_b2 = time_b.reshape(1, _C)
    lin_b2 = lin_b.reshape(1, _C)
    ln2_g2 = ln2_g.reshape(1, _C)
    ln2_b2 = ln2_b.reshape(1, _C)
    return _final(acc, y_sw, degp, t, gc_b2, time_W, time_b2,
                  lin_W, lin_b2, ln2_g2, ln2_b2)
